# Initial kernel scaffold; baseline (speedup 1.0000x reference)
import jax
import jax.numpy as jnp
from jax.experimental import pallas as pl


def kernel(data_t, data_msg, src, dst, neg_dst, n_id, t, msg, edge_index, e_id, memory, last_update, W_t, b_t, Wq, bq, Wk, bk, Wv, bv, We, Wskip, bskip, W_mlp, b_mlp, W_ls, b_ls, W_ld, b_ld, W_lf, b_lf):
    raise NotImplementedError("write your pallas kernel here")


# speedup vs baseline: 16.0041x; 16.0041x over previous
import functools

import jax
import jax.numpy as jnp
from jax import lax
from jax.experimental import pallas as pl
from jax.experimental.pallas import tpu as pltpu
from jax.experimental.pallas import tpu_sc as plsc

NUM_NODES = 100000
NUM_EVENTS = 200000
B = 4096
NL = 3 * B
SIZE = 10
E = NL * SIZE
RAW = 16
D = 128
HD = 64
HIDDEN = 64
PK = 144

NC = 2
NS = 16
NW = NC * NS

ROWS_N = NL // NW
ROWS_E = E // NW
CH = 128
NCH_N = ROWS_N // CH
NCH_E = ROWS_E // CH

f32 = jnp.float32
i32 = jnp.int32


def _mesh():
    return plsc.VectorSubcoreMesh(core_axis_name="c", subcore_axis_name="s", num_cores=2, num_subcores=16)


_SC_PARAMS = pltpu.CompilerParams(use_tc_tiling_on_sc=False,
                                  needs_layout_passes=False)


def _wid():
    return lax.axis_index("s") * NC + lax.axis_index("c")


EVW = 32


@functools.partial(
    pl.kernel,
    out_type=(
        jax.ShapeDtypeStruct((NL, D), f32),
        jax.ShapeDtypeStruct((NL, 16), i32),
        jax.ShapeDtypeStruct((E, EVW), i32),
    ),
    mesh=_mesh(),
    compiler_params=_SC_PARAMS,
    scratch_types=[
        pltpu.VMEM((CH,), i32),
        pltpu.VMEM((CH,), i32),
        pltpu.VMEM((CH, D), f32),
        pltpu.VMEM((CH, 16), i32),
        pltpu.VMEM((CH, EVW), i32),
        pltpu.SemaphoreType.DMA,
    ],
)
def _sc_gather_a(nid_h, eid_h, mem_h, lu16_h, ev32_h,
                 z_o, lu_o, ev_o,
                 nidx_v, eidx_v, zrows_v, lurows_v, evrows_v, sem):
    w = _wid()
    base_n = w * ROWS_N
    base_e = w * ROWS_E

    def nbody(i, _):
        off = base_n + i * CH
        pltpu.sync_copy(nid_h.at[pl.ds(off, CH)], nidx_v)
        pltpu.async_copy(mem_h.at[nidx_v], zrows_v, sem).wait()
        pltpu.sync_copy(zrows_v, z_o.at[pl.ds(off, CH)])
        pltpu.async_copy(lu16_h.at[nidx_v], lurows_v, sem).wait()
        pltpu.sync_copy(lurows_v, lu_o.at[pl.ds(off, CH)])
        return 0

    lax.fori_loop(0, NCH_N, nbody, 0)

    def ebody(i, _):
        off = base_e + i * CH
        pltpu.sync_copy(eid_h.at[pl.ds(off, CH)], eidx_v)
        pltpu.async_copy(ev32_h.at[eidx_v], evrows_v, sem).wait()
        pltpu.sync_copy(evrows_v, ev_o.at[pl.ds(off, CH)])
        return 0

    lax.fori_loop(0, NCH_E, ebody, 0)


@functools.partial(
    pl.kernel,
    out_type=jax.ShapeDtypeStruct((NL,), i32),
    mesh=_mesh(),
    compiler_params=_SC_PARAMS,
    scratch_types=[
        pltpu.VMEM((NL,), i32),
        pltpu.VMEM((NUM_NODES,), i32),
        pltpu.VMEM((NL,), i32),
    ],
)
def _sc_assoc(nid_h, map3_o, nid_v, table_v, map3_v):
    w = _wid()

    @pl.when(w == 0)
    def _():
        pltpu.sync_copy(nid_h, nid_v)
        lanes = lax.iota(i32, 16)
        rank = (lanes + 15) & 15

        def scat(cc, _):
            idxv = nid_v[pl.ds(cc * 16, 16)]
            comb = idxv * 16384 + (lanes + cc * 16)
            combs = jnp.sort(comb)
            key = lax.shift_right_logical(combs, 14).astype(i32)
            val = combs & 16383
            _, key_next = plsc.sort_key_val(rank, key)
            keep = (lanes == 15) | (key != key_next)
            plsc.store_scatter(table_v, [key], val, mask=keep)
            return 0

        lax.fori_loop(0, NL // 16, scat, 0)

        def gat(cc, _):
            idxv = nid_v[pl.ds(cc * 16, 16)]
            map3_v[pl.ds(cc * 16, 16)] = plsc.load_gather(table_v, [idxv])
            return 0

        lax.fori_loop(0, NL // 16, gat, 0)
        pltpu.sync_copy(map3_v, map3_o)


KVW = 2 * D + 16


@functools.partial(
    pl.kernel,
    out_type=(
        jax.ShapeDtypeStruct((E, D), f32),
        jax.ShapeDtypeStruct((E, KVW), f32),
    ),
    mesh=_mesh(),
    compiler_params=_SC_PARAMS,
    scratch_types=[
        pltpu.VMEM((CH,), i32),
        pltpu.VMEM((CH,), i32),
        pltpu.VMEM((CH, D), f32),
        pltpu.VMEM((CH, KVW), f32),
        pltpu.SemaphoreType.DMA,
    ],
)
def _sc_gather_c(dst_h, src_h, q_h, kv_h,
                 qd_o, kvs_o,
                 didx_v, sidx_v, qrows_v, kvrows_v, sem):
    w = _wid()
    base_e = w * ROWS_E

    def body(i, _):
        off = base_e + i * CH
        pltpu.sync_copy(dst_h.at[pl.ds(off, CH)], didx_v)
        pltpu.sync_copy(src_h.at[pl.ds(off, CH)], sidx_v)
        pltpu.async_copy(q_h.at[didx_v], qrows_v, sem).wait()
        pltpu.sync_copy(qrows_v, qd_o.at[pl.ds(off, CH)])
        pltpu.async_copy(kv_h.at[sidx_v], kvrows_v, sem).wait()
        pltpu.sync_copy(kvrows_v, kvs_o.at[pl.ds(off, CH)])
        return 0

    lax.fori_loop(0, NCH_E, body, 0)


@functools.partial(
    pl.kernel,
    out_type=jax.ShapeDtypeStruct((NC, NL, PK), f32),
    mesh=_mesh(),
    compiler_params=_SC_PARAMS,
    scratch_types=[
        pltpu.VMEM((CH,), i32),
        pltpu.VMEM((CH, PK), f32),
        pltpu.VMEM_SHARED((NL, PK), f32),
        pltpu.SemaphoreType.DMA,
    ],
)
def _sc_scatter_e(dst_h, wvae_h, zero_h, num_o, didx_v, rows_v, acc_sh, sem):
    c = lax.axis_index("c")
    s = lax.axis_index("s")
    w = s * NC + c
    zrows = NL // NS
    pltpu.sync_copy(zero_h.at[pl.ds(s * zrows, zrows)],
                    acc_sh.at[pl.ds(s * zrows, zrows)])
    plsc.subcore_barrier()

    base_e = w * ROWS_E

    def body(i, _):
        off = base_e + i * CH
        pltpu.sync_copy(dst_h.at[pl.ds(off, CH)], didx_v)
        pltpu.sync_copy(wvae_h.at[pl.ds(off, CH)], rows_v)
        pltpu.sync_copy(rows_v, acc_sh.at[didx_v], add=True)
        return 0

    lax.fori_loop(0, NCH_E, body, 0)
    plsc.subcore_barrier()
    pltpu.sync_copy(acc_sh.at[pl.ds(s * zrows, zrows)],
                    num_o.at[c, pl.ds(s * zrows, zrows)])


@functools.partial(
    pl.kernel,
    out_type=jax.ShapeDtypeStruct((NL, HIDDEN), f32),
    mesh=_mesh(),
    compiler_params=_SC_PARAMS,
    scratch_types=[
        pltpu.VMEM((CH,), i32),
        pltpu.VMEM((CH, HIDDEN), f32),
        pltpu.SemaphoreType.DMA,
    ],
)
def _sc_gather_g(map3_h, h_h, hg_o, idx_v, rows_v, sem):
    w = _wid()
    base = w * ROWS_N

    def body(i, _):
        off = base + i * CH
        pltpu.sync_copy(map3_h.at[pl.ds(off, CH)], idx_v)
        pltpu.async_copy(h_h.at[idx_v], rows_v, sem).wait()
        pltpu.sync_copy(rows_v, hg_o.at[pl.ds(off, CH)])
        return 0

    lax.fori_loop(0, NCH_N, body, 0)


BLK_N = 512
BLK_E = 1024


def _tc_qkv_body(z_r, lu_r, wq_r, bq_r, wk_r, bk_r, wv_r, bv_r, q_o, kv_o):
    z = z_r[...]
    q_o[...] = jnp.dot(z, wq_r[...], preferred_element_type=f32) + bq_r[...]
    k = jnp.dot(z, wk_r[...], preferred_element_type=f32) + bk_r[...]
    v = jnp.dot(z, wv_r[...], preferred_element_type=f32) + bv_r[...]
    lub = lax.bitcast_convert_type(lu_r[...], f32)
    kv_o[...] = jnp.concatenate([k, v, lub], axis=1)


def _tc_qkv(z, lu, Wq, bq, Wk, bk, Wv, bv):
    full = lambda shape: pl.BlockSpec(shape, lambda i: (0,) * len(shape))
    return pl.pallas_call(
        _tc_qkv_body,
        grid=(NL // BLK_N,),
        in_specs=[
            pl.BlockSpec((BLK_N, D), lambda i: (i, 0)),
            pl.BlockSpec((BLK_N, 16), lambda i: (i, 0)),
            full((D, D)), full((1, D)), full((D, D)), full((1, D)),
            full((D, D)), full((1, D)),
        ],
        out_specs=[
            pl.BlockSpec((BLK_N, D), lambda i: (i, 0)),
            pl.BlockSpec((BLK_N, KVW), lambda i: (i, 0)),
        ],
        out_shape=[
            jax.ShapeDtypeStruct((NL, D), f32),
            jax.ShapeDtypeStruct((NL, KVW), f32),
        ],
    )(z, lu, Wq, bq, Wk, bk, Wv, bv)


def _tc_edge_body(qd_r, kvs_r, ev_r, wt_r, bt_r, wet_r, wem_r, wvae_o):
    kvs = kvs_r[...]
    evr = ev_r[...]
    lus = lax.bitcast_convert_type(kvs[:, 2 * D:2 * D + 1], i32)
    te = evr[:, 0:1]
    dm = lax.bitcast_convert_type(evr[:, 1:1 + RAW], f32)
    rel_t = (lus - te).astype(f32)
    enc = jnp.cos(rel_t * wt_r[...] + bt_r[...])
    ev = (jnp.dot(enc, wet_r[...], preferred_element_type=f32)
          + jnp.dot(dm, wem_r[...], preferred_element_type=f32))
    qd = qd_r[...]
    ke = kvs[:, :D] + ev
    ve = kvs[:, D:2 * D] + ev
    prod = qd * ke
    a0 = jnp.sum(prod[:, :HD], axis=1, keepdims=True) * (1.0 / 8.0)
    a1 = jnp.sum(prod[:, HD:], axis=1, keepdims=True) * (1.0 / 8.0)
    ae0 = jnp.exp(a0)
    ae1 = jnp.exp(a1)
    wv = jnp.concatenate([ae0 * ve[:, :HD], ae1 * ve[:, HD:]], axis=1)
    pad = jnp.zeros((wv.shape[0], PK - D - 2), dtype=f32)
    wvae_o[...] = jnp.concatenate([wv, ae0, ae1, pad], axis=1)


def _tc_edge(qd, kvs, ev, W_t, b_t, We_t, We_m):
    full = lambda shape: pl.BlockSpec(shape, lambda i: (0,) * len(shape))
    return pl.pallas_call(
        _tc_edge_body,
        grid=(E // BLK_E,),
        in_specs=[
            pl.BlockSpec((BLK_E, D), lambda i: (i, 0)),
            pl.BlockSpec((BLK_E, KVW), lambda i: (i, 0)),
            pl.BlockSpec((BLK_E, EVW), lambda i: (i, 0)),
            full((1, D)), full((1, D)), full((D, D)), full((RAW, D)),
        ],
        out_specs=pl.BlockSpec((BLK_E, PK), lambda i: (i, 0)),
        out_shape=jax.ShapeDtypeStruct((E, PK), f32),
    )(qd, kvs, ev, W_t, b_t, We_t, We_m)


def _tc_out_body(num_r, z_r, wskip_r, bskip_r, wmlp_r, bmlp_r, h_o):
    n = num_r[0] + num_r[1]
    s0 = n[:, D:D + 1] + 1e-16
    s1 = n[:, D + 1:D + 2] + 1e-16
    out = jnp.concatenate([n[:, :HD] / s0, n[:, HD:D] / s1], axis=1)
    out = out + jnp.dot(z_r[...], wskip_r[...],
                        preferred_element_type=f32) + bskip_r[...]
    h_o[...] = jnp.dot(out, wmlp_r[...],
                       preferred_element_type=f32) + bmlp_r[...]


def _tc_out(num2, z, Wskip, bskip, W_mlp, b_mlp):
    full = lambda shape: pl.BlockSpec(shape, lambda i: (0,) * len(shape))
    return pl.pallas_call(
        _tc_out_body,
        grid=(NL // BLK_N,),
        in_specs=[
            pl.BlockSpec((NC, BLK_N, PK), lambda i: (0, i, 0)),
            pl.BlockSpec((BLK_N, D), lambda i: (i, 0)),
            full((D, D)), full((1, D)), full((D, HIDDEN)), full((1, HIDDEN)),
        ],
        out_specs=pl.BlockSpec((BLK_N, HIDDEN), lambda i: (i, 0)),
        out_shape=jax.ShapeDtypeStruct((NL, HIDDEN), f32),
    )(num2, z, Wskip, bskip, W_mlp, b_mlp)


def _tc_link_body(hg_r, wls_r, bls_r, wld_r, bld_r, wlf_r, blf_r,
                  pos_o, neg_o):
    hg = hg_r[...]
    zs = hg[:B]
    zd = hg[B:2 * B]
    zn = hg[2 * B:]
    a = jnp.dot(zs, wls_r[...], preferred_element_type=f32) + bls_r[...]
    hp = jnp.maximum(a + jnp.dot(zd, wld_r[...],
                                 preferred_element_type=f32) + bld_r[...], 0.0)
    hn = jnp.maximum(a + jnp.dot(zn, wld_r[...],
                                 preferred_element_type=f32) + bld_r[...], 0.0)
    pos_o[...] = jnp.dot(hp, wlf_r[...], preferred_element_type=f32) + blf_r[...]
    neg_o[...] = jnp.dot(hn, wlf_r[...], preferred_element_type=f32) + blf_r[...]


def _tc_link(hg, W_ls, b_ls, W_ld, b_ld, W_lf, b_lf):
    full = lambda shape: pl.BlockSpec(shape, lambda: (0,) * len(shape))
    return pl.pallas_call(
        _tc_link_body,
        in_specs=[
            full((NL, HIDDEN)),
            full((HIDDEN, HIDDEN)), full((1, HIDDEN)),
            full((HIDDEN, HIDDEN)), full((1, HIDDEN)),
            full((HIDDEN, 1)), full((1, 1)),
        ],
        out_specs=[full((B, 1)), full((B, 1))],
        out_shape=[
            jax.ShapeDtypeStruct((B, 1), f32),
            jax.ShapeDtypeStruct((B, 1), f32),
        ],
    )(hg, W_ls, b_ls, W_ld, b_ld, W_lf, b_lf)


def kernel(data_t, data_msg, src, dst, neg_dst, n_id, t, msg, edge_index, e_id,
           memory, last_update, W_t, b_t, Wq, bq, Wk, bk, Wv, bv, We, Wskip,
           bskip, W_mlp, b_mlp, W_ls, b_ls, W_ld, b_ld, W_lf, b_lf):
    src_l = edge_index[0]
    dst_l = edge_index[1]
    row = lambda x: x.reshape(1, -1)
    lu16 = jnp.broadcast_to(last_update[:, None], (NUM_NODES, 16))
    ev32 = jnp.concatenate(
        [data_t[:, None], lax.bitcast_convert_type(data_msg, i32),
         jnp.zeros((NUM_EVENTS, EVW - 1 - RAW), dtype=i32)], axis=1)

    z, lu, ev = _sc_gather_a(n_id, e_id, memory, lu16, ev32)
    map3 = _sc_assoc(n_id)
    q, kv = _tc_qkv(z, lu, Wq, row(bq), Wk, row(bk), Wv, row(bv))
    qd, kvs = _sc_gather_c(dst_l, src_l, q, kv)
    wvae = _tc_edge(qd, kvs, ev, W_t, row(b_t), We[:D], We[D:])
    zero = jnp.zeros((NL, PK), dtype=f32)
    num2 = _sc_scatter_e(dst_l, wvae, zero)
    h = _tc_out(num2, z, Wskip, row(bskip), W_mlp, row(b_mlp))
    hg = _sc_gather_g(map3, h)
    pos_out, neg_out = _tc_link(hg, W_ls, row(b_ls), W_ld, row(b_ld),
                                W_lf, row(b_lf))
    return pos_out, neg_out
